# Initial kernel scaffold; baseline (speedup 1.0000x reference)
import jax
import jax.numpy as jnp
from jax.experimental import pallas as pl


def kernel(x, pos, edge_index, We1, be1, We2, be2, Wn1, bn1, Wn2, bn2, gamma, beta):
    raise NotImplementedError("write your pallas kernel here")


# speedup vs baseline: 4.0744x; 4.0744x over previous
import functools

import jax
import jax.numpy as jnp
from jax import lax
from jax.experimental import pallas as pl
from jax.experimental.pallas import tpu as pltpu
from jax.experimental.pallas import tpu_sc as plsc

H = 128
NC = 2
NS = 16
NW = NC * NS
L = 16


def _dot_t(a, w):
    return lax.dot_general(a, w, (((1,), (1,)), ((), ())),
                           preferred_element_type=jnp.float32)


def _pre_kernel(x_ref, a_ref, b_ref, be1_ref, xa_ref, xb_ref):
    xv = x_ref[...]
    xa_ref[...] = _dot_t(xv, a_ref[...])
    xb_ref[...] = _dot_t(xv, b_ref[...]) + be1_ref[...]


def _make_gather_kernel(n_nodes, n_edges, eb):
    ew = n_edges // NW
    nb = ew // eb
    nh = nb // 2
    assert nb == 2 * nh + 1
    ng = eb // L
    nk = H // L
    mesh = plsc.VectorSubcoreMesh(core_axis_name="c", subcore_axis_name="s",
                                  num_cores=NC, num_subcores=NS)

    @functools.partial(
        pl.kernel,
        out_type=jax.ShapeDtypeStruct((n_edges, H), jnp.float32),
        mesh=mesh,
        scratch_types=[
            pltpu.VMEM((n_nodes,), jnp.float32),
            pltpu.VMEM((n_nodes,), jnp.float32),
            pltpu.VMEM((n_nodes,), jnp.float32),
            pltpu.VMEM((H,), jnp.float32),
            pltpu.VMEM((eb,), jnp.int32),
            pltpu.VMEM((eb,), jnp.int32),
            pltpu.VMEM((eb,), jnp.int32),
            pltpu.VMEM((eb,), jnp.int32),
            pltpu.VMEM((eb, H), jnp.float32),
            pltpu.VMEM((eb, H), jnp.float32),
            pltpu.VMEM((eb, H), jnp.float32),
            pltpu.VMEM((eb, H), jnp.float32),
            pltpu.VMEM((eb,), jnp.float32),
            pltpu.VMEM((eb,), jnp.float32),
            pltpu.SemaphoreType.DMA,
            pltpu.SemaphoreType.DMA,
        ],
        compiler_params=pltpu.CompilerParams(needs_layout_passes=False),
    )
    def gather_kernel(xa_hbm, xb_hbm, px_hbm, py_hbm, pz_hbm, c_hbm,
                      row_hbm, col_hbm, s_hbm,
                      pxv, pyv, pzv, cv, ri0, ci0, ri1, ci1,
                      a0, b0, a1, b1, d0, d1, g0, g1):
        wid = lax.axis_index("s") * NC + lax.axis_index("c")
        base = wid * ew

        pltpu.sync_copy(px_hbm, pxv)
        pltpu.sync_copy(py_hbm, pyv)
        pltpu.sync_copy(pz_hbm, pzv)
        pltpu.sync_copy(c_hbm, cv)
        cvs = [cv[pl.ds(k * L, L)] for k in range(nk)]

        def dist2_block(ri, ci, dbuf):
            for g in range(ng):
                sl = pl.ds(g * L, L)
                r16 = ri[sl]
                c16 = ci[sl]
                dx = (plsc.load_gather(pxv, [r16])
                      - plsc.load_gather(pxv, [c16]))
                dy = (plsc.load_gather(pyv, [r16])
                      - plsc.load_gather(pyv, [c16]))
                dz = (plsc.load_gather(pzv, [r16])
                      - plsc.load_gather(pzv, [c16]))
                dbuf[sl] = dx * dx + dy * dy + dz * dz

        def fire(j, ri, ci, abuf, bbuf, dbuf, sem):
            off = base + j * eb
            pltpu.sync_copy(row_hbm.at[pl.ds(off, eb)], ri)
            pltpu.sync_copy(col_hbm.at[pl.ds(off, eb)], ci)
            pltpu.async_copy(xa_hbm.at[ri], abuf, sem)
            pltpu.async_copy(xb_hbm.at[ci], bbuf, sem)
            dist2_block(ri, ci, dbuf)

        def drain(j, abuf, bbuf, dbuf, ri, ci, sem):
            off = base + j * eb
            pltpu.make_async_copy(xa_hbm.at[ri], abuf, sem).wait()
            pltpu.make_async_copy(xb_hbm.at[ci], bbuf, sem).wait()

            def fold(e, _):
                dv = plsc.load_gather(dbuf, [jnp.full((L,), e, jnp.int32)])
                for k in range(nk):
                    sl = pl.ds(k * L, L)
                    abuf[e, sl] = abuf[e, sl] + bbuf[e, sl] + dv * cvs[k]
                return 0

            lax.fori_loop(0, eb, fold, 0)
            pltpu.sync_copy(abuf, s_hbm.at[pl.ds(off, eb)])

        fire(0, ri0, ci0, a0, b0, d0, g0)

        def body(jj, _):
            a = 2 * jj
            fire(a + 1, ri1, ci1, a1, b1, d1, g1)
            drain(a, a0, b0, d0, ri0, ci0, g0)
            fire(a + 2, ri0, ci0, a0, b0, d0, g0)
            drain(a + 1, a1, b1, d1, ri1, ci1, g1)
            return 0

        lax.fori_loop(0, nh, body, 0)
        drain(nb - 1, a0, b0, d0, ri0, ci0, g0)

    return gather_kernel


def _edge_mlp_kernel(s_ref, w2_ref, be2_ref, ef_ref):
    hpre = s_ref[...]
    h = hpre * lax.logistic(hpre)
    e = _dot_t(h, w2_ref[...]) + be2_ref[...]
    ef_ref[...] = e * lax.logistic(e)


def _make_scatter_kernel(n_nodes, n_edges, eb):
    ew = n_edges // NW
    nb = ew // eb
    nh = nb // 2
    assert nb == 2 * nh + 1
    rows_per_tile = n_nodes // NS
    zc = 5
    zrows = rows_per_tile // zc
    mesh = plsc.VectorSubcoreMesh(core_axis_name="c", subcore_axis_name="s",
                                  num_cores=NC, num_subcores=NS)

    @functools.partial(
        pl.kernel,
        out_type=jax.ShapeDtypeStruct((NC * n_nodes, H), jnp.float32),
        mesh=mesh,
        scratch_types=[
            pltpu.VMEM((eb,), jnp.int32),
            pltpu.VMEM((eb,), jnp.int32),
            pltpu.VMEM((eb, H), jnp.float32),
            pltpu.VMEM((eb, H), jnp.float32),
            pltpu.VMEM((zrows, H), jnp.float32),
            pltpu.VMEM_SHARED((n_nodes, H), jnp.float32),
            pltpu.SemaphoreType.DMA,
            pltpu.SemaphoreType.DMA,
        ],
        compiler_params=pltpu.CompilerParams(use_tc_tiling_on_sc=False),
    )
    def scatter_kernel(ef_hbm, row_hbm, zeros_hbm, out_hbm,
                       i0, i1, e0, e1, zbuf, agg_s, l0, l1):
        c = lax.axis_index("c")
        s = lax.axis_index("s")
        tile_row0 = s * rows_per_tile

        def zinit(k, _):
            r0 = tile_row0 + k * zrows
            pltpu.sync_copy(zeros_hbm.at[pl.ds(r0, zrows)], zbuf)
            pltpu.sync_copy(zbuf, agg_s.at[pl.ds(r0, zrows)])
            return 0

        lax.fori_loop(0, zc, zinit, 0)
        plsc.subcore_barrier()

        base = (c * NS + s) * ew

        def fire(j, idxb, ebuf, sem):
            off = base + j * eb
            ci = pltpu.async_copy(row_hbm.at[pl.ds(off, eb)], idxb, sem)
            ce = pltpu.async_copy(ef_hbm.at[pl.ds(off, eb)], ebuf, sem)
            return ci, ce

        def drain(idxb, ebuf, ci, ce):
            ci.wait()
            ce.wait()
            pltpu.sync_copy(ebuf, agg_s.at[idxb], add=True)

        fire(0, i0, e0, l0)

        def body(jj, _):
            a = 2 * jj
            fire(a + 1, i1, e1, l1)
            drain(i0, e0,
                  pltpu.make_async_copy(row_hbm.at[pl.ds(0, eb)], i0, l0),
                  pltpu.make_async_copy(ef_hbm.at[pl.ds(0, eb)], e0, l0))
            fire(a + 2, i0, e0, l0)
            drain(i1, e1,
                  pltpu.make_async_copy(row_hbm.at[pl.ds(0, eb)], i1, l1),
                  pltpu.make_async_copy(ef_hbm.at[pl.ds(0, eb)], e1, l1))
            return 0

        lax.fori_loop(0, nh, body, 0)
        drain(i0, e0,
              pltpu.make_async_copy(row_hbm.at[pl.ds(0, eb)], i0, l0),
              pltpu.make_async_copy(ef_hbm.at[pl.ds(0, eb)], e0, l0))

        plsc.subcore_barrier()

        def zout(k, _):
            r0 = tile_row0 + k * zrows
            pltpu.sync_copy(agg_s.at[pl.ds(r0, zrows)], zbuf)
            pltpu.sync_copy(zbuf, out_hbm.at[pl.ds(c * n_nodes + r0, zrows)])
            return 0

        lax.fori_loop(0, zc, zout, 0)

    return scatter_kernel


def _node_kernel(x_ref, a0_ref, a1_ref, wa_ref, wb_ref, bn1_ref, w2_ref,
                 bn2_ref, g_ref, b_ref, out_ref):
    xv = x_ref[...]
    agg = a0_ref[...] + a1_ref[...]
    u = _dot_t(xv, wa_ref[...]) + _dot_t(agg, wb_ref[...]) + bn1_ref[...]
    u = u * lax.logistic(u)
    upd = _dot_t(u, w2_ref[...]) + bn2_ref[...]
    y = xv + upd
    mean = jnp.mean(y, axis=1, keepdims=True)
    yc = y - mean
    var = jnp.mean(yc * yc, axis=1, keepdims=True)
    out_ref[...] = yc * lax.rsqrt(var + 1e-5) * g_ref[...] + b_ref[...]


def kernel(x, pos, edge_index, We1, be1, We2, be2, Wn1, bn1, Wn2, bn2,
           gamma, beta):
    n_nodes = x.shape[0]
    n_edges = edge_index.shape[1]

    row = edge_index[0]
    col = edge_index[1]

    a_w = We1[:, :H]
    b_w = We1[:, H:2 * H]
    c_w = We1[:, 2 * H].reshape(1, H)
    be1r = be1.reshape(1, H)
    be2r = be2.reshape(1, H)
    wn1a = Wn1[:, :H]
    wn1b = Wn1[:, H:]
    bn1r = bn1.reshape(1, H)
    bn2r = bn2.reshape(1, H)
    gr = gamma.reshape(1, H)
    br = beta.reshape(1, H)
    px = pos[:, 0]
    py = pos[:, 1]
    pz = pos[:, 2]

    nblk = 2000
    xa, xb = pl.pallas_call(
        _pre_kernel,
        grid=(n_nodes // nblk,),
        in_specs=[
            pl.BlockSpec((nblk, H), lambda i: (i, 0)),
            pl.BlockSpec((H, H), lambda i: (0, 0)),
            pl.BlockSpec((H, H), lambda i: (0, 0)),
            pl.BlockSpec((1, H), lambda i: (0, 0)),
        ],
        out_specs=[
            pl.BlockSpec((nblk, H), lambda i: (i, 0)),
            pl.BlockSpec((nblk, H), lambda i: (i, 0)),
        ],
        out_shape=[
            jax.ShapeDtypeStruct((n_nodes, H), jnp.float32),
            jax.ShapeDtypeStruct((n_nodes, H), jnp.float32),
        ],
    )(x, a_w, b_w, be1r)

    s = _make_gather_kernel(n_nodes, n_edges, 80)(
        xa, xb, px, py, pz, c_w.reshape(H), row, col)

    eblk = 2000
    ef = pl.pallas_call(
        _edge_mlp_kernel,
        grid=(n_edges // eblk,),
        in_specs=[
            pl.BlockSpec((eblk, H), lambda i: (i, 0)),
            pl.BlockSpec((H, H), lambda i: (0, 0)),
            pl.BlockSpec((1, H), lambda i: (0, 0)),
        ],
        out_specs=pl.BlockSpec((eblk, H), lambda i: (i, 0)),
        out_shape=jax.ShapeDtypeStruct((n_edges, H), jnp.float32),
    )(s, We2, be2r)

    zeros_nh = jnp.zeros((n_nodes, H), jnp.float32)
    aggp = _make_scatter_kernel(n_nodes, n_edges, 80)(ef, row, zeros_nh)

    nb2 = n_nodes // nblk
    out = pl.pallas_call(
        _node_kernel,
        grid=(nb2,),
        in_specs=[
            pl.BlockSpec((nblk, H), lambda i: (i, 0)),
            pl.BlockSpec((nblk, H), lambda i: (i, 0)),
            pl.BlockSpec((nblk, H), lambda i, n=nb2: (i + n, 0)),
            pl.BlockSpec((H, H), lambda i: (0, 0)),
            pl.BlockSpec((H, H), lambda i: (0, 0)),
            pl.BlockSpec((1, H), lambda i: (0, 0)),
            pl.BlockSpec((H, H), lambda i: (0, 0)),
            pl.BlockSpec((1, H), lambda i: (0, 0)),
            pl.BlockSpec((1, H), lambda i: (0, 0)),
            pl.BlockSpec((1, H), lambda i: (0, 0)),
        ],
        out_specs=pl.BlockSpec((nblk, H), lambda i: (i, 0)),
        out_shape=jax.ShapeDtypeStruct((n_nodes, H), jnp.float32),
    )(x, aggp, aggp, wn1a, wn1b, bn1r, Wn2, bn2r, gr, br)

    return out


# speedup vs baseline: 4.7750x; 1.1719x over previous
import functools

import jax
import jax.numpy as jnp
from jax import lax
from jax.experimental import pallas as pl
from jax.experimental.pallas import tpu as pltpu
from jax.experimental.pallas import tpu_sc as plsc

H = 128
NC = 2
NS = 16
NW = NC * NS
L = 16


def _dot_t(a, w):
    return lax.dot_general(a, w, (((1,), (1,)), ((), ())),
                           preferred_element_type=jnp.float32)


def _pre_kernel(x_ref, a_ref, b_ref, be1_ref, xa_ref, xb_ref):
    xv = x_ref[...]
    xa_ref[...] = _dot_t(xv, a_ref[...])
    xb_ref[...] = _dot_t(xv, b_ref[...]) + be1_ref[...]


def _make_gather_kernel(n_nodes, n_edges, eb):
    ew = n_edges // NW
    nb = ew // eb
    ng = eb // L
    nk = H // L
    nd = 3
    nsteady = (nb - nd) // nd
    ntail = nb - nd - nsteady * nd
    mesh = plsc.VectorSubcoreMesh(core_axis_name="c", subcore_axis_name="s",
                                  num_cores=NC, num_subcores=NS)

    @functools.partial(
        pl.kernel,
        out_type=jax.ShapeDtypeStruct((n_edges, H), jnp.float32),
        mesh=mesh,
        scratch_types=[
            pltpu.VMEM((n_nodes,), jnp.float32),
            pltpu.VMEM((n_nodes,), jnp.float32),
            pltpu.VMEM((n_nodes,), jnp.float32),
            pltpu.VMEM((H,), jnp.float32),
            pltpu.VMEM((nb, eb), jnp.int32),
            pltpu.VMEM((nb, eb), jnp.int32),
            [pltpu.VMEM((eb, H), jnp.float32) for _ in range(nd)],
            [pltpu.VMEM((eb, H), jnp.float32) for _ in range(nd)],
            [pltpu.VMEM((eb,), jnp.float32) for _ in range(nd)],
            [pltpu.SemaphoreType.DMA for _ in range(nd)],
            pltpu.SemaphoreType.DMA,
        ],
        compiler_params=pltpu.CompilerParams(needs_layout_passes=False),
    )
    def gather_kernel(xa_hbm, xb_hbm, px_hbm, py_hbm, pz_hbm, c_hbm,
                      row3_hbm, col3_hbm, s_hbm,
                      pxv, pyv, pzv, cv, riv, civ,
                      abufs, bbufs, dbufs, gsems, psem):
        wid = lax.axis_index("s") * NC + lax.axis_index("c")
        base = wid * ew

        pltpu.sync_copy(px_hbm, pxv)
        pltpu.sync_copy(py_hbm, pyv)
        pltpu.sync_copy(pz_hbm, pzv)
        pltpu.sync_copy(c_hbm, cv)
        pltpu.sync_copy(row3_hbm.at[wid], riv)
        pltpu.sync_copy(col3_hbm.at[wid], civ)
        cvs = [cv[pl.ds(k * L, L)] for k in range(nk)]

        def dist2_block(j, dbuf):
            for g in range(ng):
                sl = pl.ds(g * L, L)
                r16 = riv[j, sl]
                c16 = civ[j, sl]
                dx = (plsc.load_gather(pxv, [r16])
                      - plsc.load_gather(pxv, [c16]))
                dy = (plsc.load_gather(pyv, [r16])
                      - plsc.load_gather(pyv, [c16]))
                dz = (plsc.load_gather(pzv, [r16])
                      - plsc.load_gather(pzv, [c16]))
                dbuf[sl] = dx * dx + dy * dy + dz * dz

        def fire(j, slot):
            pltpu.async_copy(xa_hbm.at[riv.at[j]], abufs[slot], gsems[slot])
            pltpu.async_copy(xb_hbm.at[civ.at[j]], bbufs[slot], gsems[slot])
            dist2_block(j, dbufs[slot])

        def drain(j, slot):
            abuf, bbuf, dbuf = abufs[slot], bbufs[slot], dbufs[slot]
            pltpu.make_async_copy(xa_hbm.at[riv.at[j]], abuf,
                                  gsems[slot]).wait()
            pltpu.make_async_copy(xb_hbm.at[civ.at[j]], bbuf,
                                  gsems[slot]).wait()

            def fold(e, _):
                dv = plsc.load_gather(dbuf, [jnp.full((L,), e, jnp.int32)])
                for k in range(nk):
                    sl = pl.ds(k * L, L)
                    abuf[e, sl] = abuf[e, sl] + bbuf[e, sl] + dv * cvs[k]
                return 0

            lax.fori_loop(0, eb, fold, 0)
            pltpu.sync_copy(abuf, s_hbm.at[pl.ds(base + j * eb, eb)])

        for p in range(nd):
            fire(p, p)

        def body(jj, _):
            j = jj * nd
            for p in range(nd):
                drain(j + p, p)
                fire(j + nd + p, p)
            return 0

        lax.fori_loop(0, nsteady, body, 0)
        jt = nsteady * nd
        for p in range(ntail):
            drain(jt + p, p)
            fire(jt + nd + p, p)
        for p in range(nd):
            q = jt + ntail + p
            drain(q, q % nd)

    return gather_kernel


def _edge_mlp_kernel(s_ref, w2_ref, be2_ref, ef_ref):
    hpre = s_ref[...]
    h = hpre * lax.logistic(hpre)
    e = _dot_t(h, w2_ref[...]) + be2_ref[...]
    ef_ref[...] = e * lax.logistic(e)


def _make_scatter_kernel(n_nodes, n_edges, eb):
    ew = n_edges // NW
    nb = ew // eb
    rows_per_tile = n_nodes // NS
    zc = 25
    zrows = rows_per_tile // zc
    mesh = plsc.VectorSubcoreMesh(core_axis_name="c", subcore_axis_name="s",
                                  num_cores=NC, num_subcores=NS)

    @functools.partial(
        pl.kernel,
        out_type=jax.ShapeDtypeStruct((NC * n_nodes, H), jnp.float32),
        mesh=mesh,
        scratch_types=[
            pltpu.VMEM((nb, eb), jnp.int32),
            [pltpu.VMEM((eb, H), jnp.float32) for _ in range(3)],
            pltpu.VMEM((zrows, H), jnp.float32),
            pltpu.VMEM_SHARED((n_nodes, H), jnp.float32),
            [pltpu.SemaphoreType.DMA for _ in range(3)],
        ],
        compiler_params=pltpu.CompilerParams(use_tc_tiling_on_sc=False),
    )
    def scatter_kernel(ef_hbm, row3_hbm, zeros_hbm, out_hbm,
                       idxw, ebufs, zbuf, agg_s, lsems):
        c = lax.axis_index("c")
        s = lax.axis_index("s")
        tile_row0 = s * rows_per_tile
        w = c * NS + s
        base = w * ew

        pltpu.sync_copy(row3_hbm.at[w], idxw)

        def zinit(k, _):
            r0 = tile_row0 + k * zrows
            pltpu.sync_copy(zeros_hbm.at[pl.ds(r0, zrows)], zbuf)
            pltpu.sync_copy(zbuf, agg_s.at[pl.ds(r0, zrows)])
            return 0

        lax.fori_loop(0, zc, zinit, 0)
        plsc.subcore_barrier()

        nd = 3
        nsteady = (nb - nd) // nd
        ntail = nb - nd - nsteady * nd

        def fire(j, slot):
            pltpu.async_copy(ef_hbm.at[pl.ds(base + j * eb, eb)],
                             ebufs[slot], lsems[slot])

        def drain(j, slot):
            pltpu.make_async_copy(ef_hbm.at[pl.ds(base + j * eb, eb)],
                                  ebufs[slot], lsems[slot]).wait()
            pltpu.sync_copy(ebufs[slot], agg_s.at[idxw.at[j]], add=True)

        for p in range(nd):
            fire(p, p)

        def body(jj, _):
            j = jj * nd
            for p in range(nd):
                drain(j + p, p)
                fire(j + nd + p, p)
            return 0

        lax.fori_loop(0, nsteady, body, 0)
        jt = nsteady * nd
        for p in range(ntail):
            drain(jt + p, p)
            fire(jt + nd + p, p)
        for p in range(nd):
            q = jt + ntail + p
            drain(q, q % nd)

        plsc.subcore_barrier()

        def zout(k, _):
            r0 = tile_row0 + k * zrows
            pltpu.sync_copy(agg_s.at[pl.ds(r0, zrows)], zbuf)
            pltpu.sync_copy(zbuf, out_hbm.at[pl.ds(c * n_nodes + r0, zrows)])
            return 0

        lax.fori_loop(0, zc, zout, 0)

    return scatter_kernel


def _node_kernel(x_ref, a0_ref, a1_ref, wa_ref, wb_ref, bn1_ref, w2_ref,
                 bn2_ref, g_ref, b_ref, out_ref):
    xv = x_ref[...]
    agg = a0_ref[...] + a1_ref[...]
    u = _dot_t(xv, wa_ref[...]) + _dot_t(agg, wb_ref[...]) + bn1_ref[...]
    u = u * lax.logistic(u)
    upd = _dot_t(u, w2_ref[...]) + bn2_ref[...]
    y = xv + upd
    mean = jnp.mean(y, axis=1, keepdims=True)
    yc = y - mean
    var = jnp.mean(yc * yc, axis=1, keepdims=True)
    out_ref[...] = yc * lax.rsqrt(var + 1e-5) * g_ref[...] + b_ref[...]


def kernel(x, pos, edge_index, We1, be1, We2, be2, Wn1, bn1, Wn2, bn2,
           gamma, beta):
    n_nodes = x.shape[0]
    n_edges = edge_index.shape[1]

    row = edge_index[0]
    col = edge_index[1]

    a_w = We1[:, :H]
    b_w = We1[:, H:2 * H]
    c_w = We1[:, 2 * H].reshape(1, H)
    be1r = be1.reshape(1, H)
    be2r = be2.reshape(1, H)
    wn1a = Wn1[:, :H]
    wn1b = Wn1[:, H:]
    bn1r = bn1.reshape(1, H)
    bn2r = bn2.reshape(1, H)
    gr = gamma.reshape(1, H)
    br = beta.reshape(1, H)
    px = pos[:, 0]
    py = pos[:, 1]
    pz = pos[:, 2]

    nblk = 2000
    xa, xb = pl.pallas_call(
        _pre_kernel,
        grid=(n_nodes // nblk,),
        in_specs=[
            pl.BlockSpec((nblk, H), lambda i: (i, 0)),
            pl.BlockSpec((H, H), lambda i: (0, 0)),
            pl.BlockSpec((H, H), lambda i: (0, 0)),
            pl.BlockSpec((1, H), lambda i: (0, 0)),
        ],
        out_specs=[
            pl.BlockSpec((nblk, H), lambda i: (i, 0)),
            pl.BlockSpec((nblk, H), lambda i: (i, 0)),
        ],
        out_shape=[
            jax.ShapeDtypeStruct((n_nodes, H), jnp.float32),
            jax.ShapeDtypeStruct((n_nodes, H), jnp.float32),
        ],
    )(x, a_w, b_w, be1r)

    eb = 80
    ew = n_edges // NW
    row3 = row.reshape(NW, ew // eb, eb)
    col3 = col.reshape(NW, ew // eb, eb)
    s = _make_gather_kernel(n_nodes, n_edges, eb)(
        xa, xb, px, py, pz, c_w.reshape(H), row3, col3)

    eblk = 4000
    ef = pl.pallas_call(
        _edge_mlp_kernel,
        grid=(n_edges // eblk,),
        in_specs=[
            pl.BlockSpec((eblk, H), lambda i: (i, 0)),
            pl.BlockSpec((H, H), lambda i: (0, 0)),
            pl.BlockSpec((1, H), lambda i: (0, 0)),
        ],
        out_specs=pl.BlockSpec((eblk, H), lambda i: (i, 0)),
        out_shape=jax.ShapeDtypeStruct((n_edges, H), jnp.float32),
    )(s, We2, be2r)

    zeros_nh = jnp.zeros((n_nodes, H), jnp.float32)
    aggp = _make_scatter_kernel(n_nodes, n_edges, eb)(ef, row3, zeros_nh)

    nb2 = n_nodes // nblk
    out = pl.pallas_call(
        _node_kernel,
        grid=(nb2,),
        in_specs=[
            pl.BlockSpec((nblk, H), lambda i: (i, 0)),
            pl.BlockSpec((nblk, H), lambda i: (i, 0)),
            pl.BlockSpec((nblk, H), lambda i, n=nb2: (i + n, 0)),
            pl.BlockSpec((H, H), lambda i: (0, 0)),
            pl.BlockSpec((H, H), lambda i: (0, 0)),
            pl.BlockSpec((1, H), lambda i: (0, 0)),
            pl.BlockSpec((H, H), lambda i: (0, 0)),
            pl.BlockSpec((1, H), lambda i: (0, 0)),
            pl.BlockSpec((1, H), lambda i: (0, 0)),
            pl.BlockSpec((1, H), lambda i: (0, 0)),
        ],
        out_specs=pl.BlockSpec((nblk, H), lambda i: (i, 0)),
        out_shape=jax.ShapeDtypeStruct((n_nodes, H), jnp.float32),
    )(x, aggp, aggp, wn1a, wn1b, bn1r, Wn2, bn2r, gr, br)

    return out


# speedup vs baseline: 8.2087x; 1.7191x over previous
import functools

import jax
import jax.numpy as jnp
from jax import lax
from jax.experimental import pallas as pl
from jax.experimental.pallas import tpu as pltpu
from jax.experimental.pallas import tpu_sc as plsc

H = 128
NC = 2
NS = 16
NW = NC * NS
L = 16


def _dot_t(a, w):
    return lax.dot_general(a, w, (((1,), (1,)), ((), ())),
                           preferred_element_type=jnp.float32)


def _pre_kernel(x_ref, a_ref, b_ref, be1_ref, xa_ref, xb_ref):
    xv = x_ref[...]
    xa_ref[...] = _dot_t(xv, a_ref[...])
    xb_ref[...] = _dot_t(xv, b_ref[...]) + be1_ref[...]


def _make_gather_kernel(n_nodes, n_edges, eb):
    ew = n_edges // NW
    nb = ew // eb
    ng = eb // L
    nk = H // L
    nd = 3
    nsteady = (nb - nd) // nd
    ntail = nb - nd - nsteady * nd
    mesh = plsc.VectorSubcoreMesh(core_axis_name="c", subcore_axis_name="s",
                                  num_cores=NC, num_subcores=NS)

    @functools.partial(
        pl.kernel,
        out_type=jax.ShapeDtypeStruct((n_edges, H), jnp.float32),
        mesh=mesh,
        scratch_types=[
            pltpu.VMEM((n_nodes,), jnp.float32),
            pltpu.VMEM((n_nodes,), jnp.float32),
            pltpu.VMEM((n_nodes,), jnp.float32),
            pltpu.VMEM((H,), jnp.float32),
            pltpu.VMEM((nb, eb), jnp.int32),
            pltpu.VMEM((nb, eb), jnp.int32),
            [pltpu.VMEM((eb, H), jnp.float32) for _ in range(nd)],
            [pltpu.VMEM((eb, H), jnp.float32) for _ in range(nd)],
            [pltpu.VMEM((eb,), jnp.float32) for _ in range(nd)],
            [pltpu.SemaphoreType.DMA for _ in range(nd)],
            pltpu.SemaphoreType.DMA,
        ],
        compiler_params=pltpu.CompilerParams(needs_layout_passes=False),
    )
    def gather_kernel(xa_hbm, xb_hbm, px_hbm, py_hbm, pz_hbm, c_hbm,
                      row3_hbm, col3_hbm, s_hbm,
                      pxv, pyv, pzv, cv, riv, civ,
                      abufs, bbufs, dbufs, gsems, psem):
        wid = lax.axis_index("s") * NC + lax.axis_index("c")
        base = wid * ew

        pltpu.sync_copy(px_hbm, pxv)
        pltpu.sync_copy(py_hbm, pyv)
        pltpu.sync_copy(pz_hbm, pzv)
        pltpu.sync_copy(c_hbm, cv)
        pltpu.sync_copy(row3_hbm.at[wid], riv)
        pltpu.sync_copy(col3_hbm.at[wid], civ)
        cvs = [cv[pl.ds(k * L, L)] for k in range(nk)]

        def dist2_block(j, dbuf):
            for g in range(ng):
                sl = pl.ds(g * L, L)
                r16 = riv[j, sl]
                c16 = civ[j, sl]
                dx = (plsc.load_gather(pxv, [r16])
                      - plsc.load_gather(pxv, [c16]))
                dy = (plsc.load_gather(pyv, [r16])
                      - plsc.load_gather(pyv, [c16]))
                dz = (plsc.load_gather(pzv, [r16])
                      - plsc.load_gather(pzv, [c16]))
                dbuf[sl] = dx * dx + dy * dy + dz * dz

        def fire(j, slot):
            pltpu.async_copy(xa_hbm.at[riv.at[j]], abufs[slot], gsems[slot])
            pltpu.async_copy(xb_hbm.at[civ.at[j]], bbufs[slot], gsems[slot])
            dist2_block(j, dbufs[slot])

        def drain(j, slot):
            abuf, bbuf, dbuf = abufs[slot], bbufs[slot], dbufs[slot]
            pltpu.make_async_copy(xa_hbm.at[riv.at[j]], abuf,
                                  gsems[slot]).wait()
            pltpu.make_async_copy(xb_hbm.at[civ.at[j]], bbuf,
                                  gsems[slot]).wait()

            @plsc.parallel_loop(0, eb, unroll=4)
            def fold(e):
                dv = plsc.load_gather(dbuf, [jnp.full((L,), e, jnp.int32)])
                for k in range(nk):
                    sl = pl.ds(k * L, L)
                    abuf[e, sl] = abuf[e, sl] + bbuf[e, sl] + dv * cvs[k]
            pltpu.sync_copy(abuf, s_hbm.at[pl.ds(base + j * eb, eb)])

        for p in range(nd):
            fire(p, p)

        def body(jj, _):
            j = jj * nd
            for p in range(nd):
                drain(j + p, p)
                fire(j + nd + p, p)
            return 0

        lax.fori_loop(0, nsteady, body, 0)
        jt = nsteady * nd
        for p in range(ntail):
            drain(jt + p, p)
            fire(jt + nd + p, p)
        for p in range(nd):
            q = jt + ntail + p
            drain(q, q % nd)

    return gather_kernel


def _edge_mlp_kernel(s_ref, w2_ref, be2_ref, ef_ref):
    hpre = s_ref[...]
    h = hpre * lax.logistic(hpre)
    e = _dot_t(h, w2_ref[...]) + be2_ref[...]
    ef_ref[...] = e * lax.logistic(e)


def _make_scatter_kernel(n_nodes, n_edges, eb):
    ew = n_edges // NW
    nb = ew // eb
    rows_per_tile = n_nodes // NS
    zc = 25
    zrows = rows_per_tile // zc
    mesh = plsc.VectorSubcoreMesh(core_axis_name="c", subcore_axis_name="s",
                                  num_cores=NC, num_subcores=NS)

    @functools.partial(
        pl.kernel,
        out_type=jax.ShapeDtypeStruct((NC * n_nodes, H), jnp.float32),
        mesh=mesh,
        scratch_types=[
            pltpu.VMEM((nb, eb), jnp.int32),
            [pltpu.VMEM((eb, H), jnp.float32) for _ in range(3)],
            pltpu.VMEM((zrows, H), jnp.float32),
            pltpu.VMEM_SHARED((n_nodes, H), jnp.float32),
            [pltpu.SemaphoreType.DMA for _ in range(3)],
        ],
        compiler_params=pltpu.CompilerParams(use_tc_tiling_on_sc=False),
    )
    def scatter_kernel(ef_hbm, row3_hbm, zeros_hbm, out_hbm,
                       idxw, ebufs, zbuf, agg_s, lsems):
        c = lax.axis_index("c")
        s = lax.axis_index("s")
        tile_row0 = s * rows_per_tile
        w = c * NS + s
        base = w * ew

        pltpu.sync_copy(row3_hbm.at[w], idxw)

        def zinit(k, _):
            r0 = tile_row0 + k * zrows
            pltpu.sync_copy(zeros_hbm.at[pl.ds(r0, zrows)], zbuf)
            pltpu.sync_copy(zbuf, agg_s.at[pl.ds(r0, zrows)])
            return 0

        lax.fori_loop(0, zc, zinit, 0)
        plsc.subcore_barrier()

        nd = 3
        nsteady = (nb - nd) // nd
        ntail = nb - nd - nsteady * nd

        def fire(j, slot):
            pltpu.async_copy(ef_hbm.at[pl.ds(base + j * eb, eb)],
                             ebufs[slot], lsems[slot])

        def drain(j, slot):
            pltpu.make_async_copy(ef_hbm.at[pl.ds(base + j * eb, eb)],
                                  ebufs[slot], lsems[slot]).wait()
            pltpu.sync_copy(ebufs[slot], agg_s.at[idxw.at[j]], add=True)

        for p in range(nd):
            fire(p, p)

        def body(jj, _):
            j = jj * nd
            for p in range(nd):
                drain(j + p, p)
                fire(j + nd + p, p)
            return 0

        lax.fori_loop(0, nsteady, body, 0)
        jt = nsteady * nd
        for p in range(ntail):
            drain(jt + p, p)
            fire(jt + nd + p, p)
        for p in range(nd):
            q = jt + ntail + p
            drain(q, q % nd)

        plsc.subcore_barrier()

        def zout(k, _):
            r0 = tile_row0 + k * zrows
            pltpu.sync_copy(agg_s.at[pl.ds(r0, zrows)], zbuf)
            pltpu.sync_copy(zbuf, out_hbm.at[pl.ds(c * n_nodes + r0, zrows)])
            return 0

        lax.fori_loop(0, zc, zout, 0)

    return scatter_kernel


def _node_kernel(x_ref, a0_ref, a1_ref, wa_ref, wb_ref, bn1_ref, w2_ref,
                 bn2_ref, g_ref, b_ref, out_ref):
    xv = x_ref[...]
    agg = a0_ref[...] + a1_ref[...]
    u = _dot_t(xv, wa_ref[...]) + _dot_t(agg, wb_ref[...]) + bn1_ref[...]
    u = u * lax.logistic(u)
    upd = _dot_t(u, w2_ref[...]) + bn2_ref[...]
    y = xv + upd
    mean = jnp.mean(y, axis=1, keepdims=True)
    yc = y - mean
    var = jnp.mean(yc * yc, axis=1, keepdims=True)
    out_ref[...] = yc * lax.rsqrt(var + 1e-5) * g_ref[...] + b_ref[...]


def kernel(x, pos, edge_index, We1, be1, We2, be2, Wn1, bn1, Wn2, bn2,
           gamma, beta):
    n_nodes = x.shape[0]
    n_edges = edge_index.shape[1]

    row = edge_index[0]
    col = edge_index[1]

    a_w = We1[:, :H]
    b_w = We1[:, H:2 * H]
    c_w = We1[:, 2 * H].reshape(1, H)
    be1r = be1.reshape(1, H)
    be2r = be2.reshape(1, H)
    wn1a = Wn1[:, :H]
    wn1b = Wn1[:, H:]
    bn1r = bn1.reshape(1, H)
    bn2r = bn2.reshape(1, H)
    gr = gamma.reshape(1, H)
    br = beta.reshape(1, H)
    px = pos[:, 0]
    py = pos[:, 1]
    pz = pos[:, 2]

    nblk = 2000
    xa, xb = pl.pallas_call(
        _pre_kernel,
        grid=(n_nodes // nblk,),
        in_specs=[
            pl.BlockSpec((nblk, H), lambda i: (i, 0)),
            pl.BlockSpec((H, H), lambda i: (0, 0)),
            pl.BlockSpec((H, H), lambda i: (0, 0)),
            pl.BlockSpec((1, H), lambda i: (0, 0)),
        ],
        out_specs=[
            pl.BlockSpec((nblk, H), lambda i: (i, 0)),
            pl.BlockSpec((nblk, H), lambda i: (i, 0)),
        ],
        out_shape=[
            jax.ShapeDtypeStruct((n_nodes, H), jnp.float32),
            jax.ShapeDtypeStruct((n_nodes, H), jnp.float32),
        ],
    )(x, a_w, b_w, be1r)

    eb = 80
    ew = n_edges // NW
    row3 = row.reshape(NW, ew // eb, eb)
    col3 = col.reshape(NW, ew // eb, eb)
    s = _make_gather_kernel(n_nodes, n_edges, eb)(
        xa, xb, px, py, pz, c_w.reshape(H), row3, col3)

    eblk = 4000
    ef = pl.pallas_call(
        _edge_mlp_kernel,
        grid=(n_edges // eblk,),
        in_specs=[
            pl.BlockSpec((eblk, H), lambda i: (i, 0)),
            pl.BlockSpec((H, H), lambda i: (0, 0)),
            pl.BlockSpec((1, H), lambda i: (0, 0)),
        ],
        out_specs=pl.BlockSpec((eblk, H), lambda i: (i, 0)),
        out_shape=jax.ShapeDtypeStruct((n_edges, H), jnp.float32),
    )(s, We2, be2r)

    zeros_nh = jnp.zeros((n_nodes, H), jnp.float32)
    aggp = _make_scatter_kernel(n_nodes, n_edges, eb)(ef, row3, zeros_nh)

    nb2 = n_nodes // nblk
    out = pl.pallas_call(
        _node_kernel,
        grid=(nb2,),
        in_specs=[
            pl.BlockSpec((nblk, H), lambda i: (i, 0)),
            pl.BlockSpec((nblk, H), lambda i: (i, 0)),
            pl.BlockSpec((nblk, H), lambda i, n=nb2: (i + n, 0)),
            pl.BlockSpec((H, H), lambda i: (0, 0)),
            pl.BlockSpec((H, H), lambda i: (0, 0)),
            pl.BlockSpec((1, H), lambda i: (0, 0)),
            pl.BlockSpec((H, H), lambda i: (0, 0)),
            pl.BlockSpec((1, H), lambda i: (0, 0)),
            pl.BlockSpec((1, H), lambda i: (0, 0)),
            pl.BlockSpec((1, H), lambda i: (0, 0)),
        ],
        out_specs=pl.BlockSpec((nblk, H), lambda i: (i, 0)),
        out_shape=jax.ShapeDtypeStruct((n_nodes, H), jnp.float32),
    )(x, aggp, aggp, wn1a, wn1b, bn1r, Wn2, bn2r, gr, br)

    return out


# speedup vs baseline: 8.5807x; 1.0453x over previous
import functools

import jax
import jax.numpy as jnp
from jax import lax
from jax.experimental import pallas as pl
from jax.experimental.pallas import tpu as pltpu
from jax.experimental.pallas import tpu_sc as plsc

H = 128
NC = 2
NS = 16
NW = NC * NS
L = 16


def _dot_t(a, w):
    return lax.dot_general(a, w, (((1,), (1,)), ((), ())),
                           preferred_element_type=jnp.float32)


def _pre_kernel(x_ref, a_ref, b_ref, be1_ref, xa_ref, xb_ref):
    xv = x_ref[...]
    xa_ref[...] = _dot_t(xv, a_ref[...])
    xb_ref[...] = _dot_t(xv, b_ref[...]) + be1_ref[...]


def _make_gather_kernel(n_nodes, n_edges, eb):
    ew = n_edges // NW
    nb = ew // eb
    ng = eb // L
    nk = H // L
    nd = 3
    nsteady = (nb - nd) // nd
    ntail = nb - nd - nsteady * nd
    mesh = plsc.VectorSubcoreMesh(core_axis_name="c", subcore_axis_name="s",
                                  num_cores=NC, num_subcores=NS)

    @functools.partial(
        pl.kernel,
        out_type=jax.ShapeDtypeStruct((n_edges, H), jnp.float32),
        mesh=mesh,
        scratch_types=[
            pltpu.VMEM((n_nodes,), jnp.float32),
            pltpu.VMEM((n_nodes,), jnp.float32),
            pltpu.VMEM((n_nodes,), jnp.float32),
            pltpu.VMEM((H,), jnp.float32),
            pltpu.VMEM((nb, eb), jnp.int32),
            pltpu.VMEM((nb, eb), jnp.int32),
            [pltpu.VMEM((eb, H), jnp.float32) for _ in range(nd)],
            [pltpu.VMEM((eb, H), jnp.float32) for _ in range(nd)],
            [pltpu.VMEM((eb,), jnp.float32) for _ in range(nd)],
            [pltpu.SemaphoreType.DMA for _ in range(nd)],
            pltpu.SemaphoreType.DMA,
        ],
        compiler_params=pltpu.CompilerParams(needs_layout_passes=False),
    )
    def gather_kernel(xa_hbm, xb_hbm, px_hbm, py_hbm, pz_hbm, c_hbm,
                      row3_hbm, col3_hbm, s_hbm,
                      pxv, pyv, pzv, cv, riv, civ,
                      abufs, bbufs, dbufs, gsems, psem):
        wid = lax.axis_index("s") * NC + lax.axis_index("c")
        base = wid * ew

        pltpu.sync_copy(px_hbm, pxv)
        pltpu.sync_copy(py_hbm, pyv)
        pltpu.sync_copy(pz_hbm, pzv)
        pltpu.sync_copy(c_hbm, cv)
        pltpu.sync_copy(row3_hbm.at[wid], riv)
        pltpu.sync_copy(col3_hbm.at[wid], civ)
        cvs = [cv[pl.ds(k * L, L)] for k in range(nk)]

        def dist2_block(j, dbuf):
            for g in range(ng):
                sl = pl.ds(g * L, L)
                r16 = riv[j, sl]
                c16 = civ[j, sl]
                dx = (plsc.load_gather(pxv, [r16])
                      - plsc.load_gather(pxv, [c16]))
                dy = (plsc.load_gather(pyv, [r16])
                      - plsc.load_gather(pyv, [c16]))
                dz = (plsc.load_gather(pzv, [r16])
                      - plsc.load_gather(pzv, [c16]))
                dbuf[sl] = dx * dx + dy * dy + dz * dz

        def fire(j, slot):
            pltpu.async_copy(xa_hbm.at[riv.at[j]], abufs[slot], gsems[slot])
            pltpu.async_copy(xb_hbm.at[civ.at[j]], bbufs[slot], gsems[slot])
            dist2_block(j, dbufs[slot])

        def drain(j, slot):
            abuf, bbuf, dbuf = abufs[slot], bbufs[slot], dbufs[slot]
            pltpu.make_async_copy(xa_hbm.at[riv.at[j]], abuf,
                                  gsems[slot]).wait()
            pltpu.make_async_copy(xb_hbm.at[civ.at[j]], bbuf,
                                  gsems[slot]).wait()

            @plsc.parallel_loop(0, eb, unroll=4)
            def fold(e):
                dv = plsc.load_gather(dbuf, [jnp.full((L,), e, jnp.int32)])
                for k in range(nk):
                    sl = pl.ds(k * L, L)
                    abuf[e, sl] = abuf[e, sl] + bbuf[e, sl] + dv * cvs[k]
            pltpu.sync_copy(abuf, s_hbm.at[pl.ds(base + j * eb, eb)])

        for p in range(nd):
            fire(p, p)

        def body(jj, _):
            j = jj * nd
            for p in range(nd):
                drain(j + p, p)
                fire(j + nd + p, p)
            return 0

        lax.fori_loop(0, nsteady, body, 0)
        jt = nsteady * nd
        for p in range(ntail):
            drain(jt + p, p)
            fire(jt + nd + p, p)
        for p in range(nd):
            q = jt + ntail + p
            drain(q, q % nd)

    return gather_kernel


def _edge_mlp_kernel(s_ref, w2_ref, be2_ref, ef_ref):
    hpre = s_ref[...]
    h = hpre * lax.logistic(hpre)
    e = _dot_t(h, w2_ref[...]) + be2_ref[...]
    ef_ref[...] = e * lax.logistic(e)


def _make_scatter_kernel(n_nodes, n_edges, eb):
    ew = n_edges // NW
    nb = ew // eb
    rows_per_tile = n_nodes // NS
    zc = 25
    zrows = rows_per_tile // zc
    mesh = plsc.VectorSubcoreMesh(core_axis_name="c", subcore_axis_name="s",
                                  num_cores=NC, num_subcores=NS)

    @functools.partial(
        pl.kernel,
        out_type=jax.ShapeDtypeStruct((NC * n_nodes, H), jnp.float32),
        mesh=mesh,
        scratch_types=[
            pltpu.VMEM((nb, eb), jnp.int32),
            [pltpu.VMEM((eb, H), jnp.float32) for _ in range(3)],
            pltpu.VMEM((zrows, H), jnp.float32),
            pltpu.VMEM_SHARED((n_nodes, H), jnp.float32),
            [pltpu.SemaphoreType.DMA for _ in range(3)],
        ],
        compiler_params=pltpu.CompilerParams(use_tc_tiling_on_sc=False),
    )
    def scatter_kernel(ef_hbm, row3_hbm, zeros_hbm, out_hbm,
                       idxw, ebufs, zbuf, agg_s, lsems):
        c = lax.axis_index("c")
        s = lax.axis_index("s")
        tile_row0 = s * rows_per_tile
        w = c * NS + s
        base = w * ew

        pltpu.sync_copy(row3_hbm.at[w], idxw)

        def zinit(k, _):
            r0 = tile_row0 + k * zrows
            pltpu.sync_copy(zeros_hbm.at[pl.ds(r0, zrows)], zbuf)
            pltpu.sync_copy(zbuf, agg_s.at[pl.ds(r0, zrows)])
            return 0

        lax.fori_loop(0, zc, zinit, 0)
        plsc.subcore_barrier()

        nd = 3
        nsteady = (nb - nd) // nd
        ntail = nb - nd - nsteady * nd

        def fire(j, slot):
            pltpu.async_copy(ef_hbm.at[pl.ds(base + j * eb, eb)],
                             ebufs[slot], lsems[slot])

        def drain(j, slot):
            pltpu.make_async_copy(ef_hbm.at[pl.ds(base + j * eb, eb)],
                                  ebufs[slot], lsems[slot]).wait()
            pltpu.sync_copy(ebufs[slot], agg_s.at[idxw.at[j]], add=True)

        for p in range(nd):
            fire(p, p)

        def body(jj, _):
            j = jj * nd
            for p in range(nd):
                drain(j + p, p)
                fire(j + nd + p, p)
            return 0

        lax.fori_loop(0, nsteady, body, 0)
        jt = nsteady * nd
        for p in range(ntail):
            drain(jt + p, p)
            fire(jt + nd + p, p)
        for p in range(nd):
            q = jt + ntail + p
            drain(q, q % nd)

        plsc.subcore_barrier()

        def zout(k, _):
            r0 = tile_row0 + k * zrows
            pltpu.sync_copy(agg_s.at[pl.ds(r0, zrows)], zbuf)
            pltpu.sync_copy(zbuf, out_hbm.at[pl.ds(c * n_nodes + r0, zrows)])
            return 0

        lax.fori_loop(0, zc, zout, 0)

    return scatter_kernel


def _node_kernel(x_ref, a0_ref, a1_ref, wa_ref, wb_ref, bn1_ref, w2_ref,
                 bn2_ref, g_ref, b_ref, out_ref):
    xv = x_ref[...]
    agg = a0_ref[...] + a1_ref[...]
    u = _dot_t(xv, wa_ref[...]) + _dot_t(agg, wb_ref[...]) + bn1_ref[...]
    u = u * lax.logistic(u)
    upd = _dot_t(u, w2_ref[...]) + bn2_ref[...]
    y = xv + upd
    mean = jnp.mean(y, axis=1, keepdims=True)
    yc = y - mean
    var = jnp.mean(yc * yc, axis=1, keepdims=True)
    out_ref[...] = yc * lax.rsqrt(var + 1e-5) * g_ref[...] + b_ref[...]


def kernel(x, pos, edge_index, We1, be1, We2, be2, Wn1, bn1, Wn2, bn2,
           gamma, beta):
    n_nodes = x.shape[0]
    n_edges = edge_index.shape[1]

    row = edge_index[0]
    col = edge_index[1]

    a_w = We1[:, :H]
    b_w = We1[:, H:2 * H]
    c_w = We1[:, 2 * H].reshape(1, H)
    be1r = be1.reshape(1, H)
    be2r = be2.reshape(1, H)
    wn1a = Wn1[:, :H]
    wn1b = Wn1[:, H:]
    bn1r = bn1.reshape(1, H)
    bn2r = bn2.reshape(1, H)
    gr = gamma.reshape(1, H)
    br = beta.reshape(1, H)
    px = pos[:, 0]
    py = pos[:, 1]
    pz = pos[:, 2]

    nblk = 2000
    xa, xb = pl.pallas_call(
        _pre_kernel,
        grid=(n_nodes // nblk,),
        in_specs=[
            pl.BlockSpec((nblk, H), lambda i: (i, 0)),
            pl.BlockSpec((H, H), lambda i: (0, 0)),
            pl.BlockSpec((H, H), lambda i: (0, 0)),
            pl.BlockSpec((1, H), lambda i: (0, 0)),
        ],
        out_specs=[
            pl.BlockSpec((nblk, H), lambda i: (i, 0)),
            pl.BlockSpec((nblk, H), lambda i: (i, 0)),
        ],
        out_shape=[
            jax.ShapeDtypeStruct((n_nodes, H), jnp.float32),
            jax.ShapeDtypeStruct((n_nodes, H), jnp.float32),
        ],
    )(x, a_w, b_w, be1r)

    eb = 80
    ew = n_edges // NW
    row3 = row.reshape(NW, ew // eb, eb)
    col3 = col.reshape(NW, ew // eb, eb)
    s = _make_gather_kernel(n_nodes, n_edges, eb)(
        xa, xb, px, py, pz, c_w.reshape(H), row3, col3)

    eblk = 8000
    ef = pl.pallas_call(
        _edge_mlp_kernel,
        grid=(n_edges // eblk,),
        in_specs=[
            pl.BlockSpec((eblk, H), lambda i: (i, 0)),
            pl.BlockSpec((H, H), lambda i: (0, 0)),
            pl.BlockSpec((1, H), lambda i: (0, 0)),
        ],
        out_specs=pl.BlockSpec((eblk, H), lambda i: (i, 0)),
        out_shape=jax.ShapeDtypeStruct((n_edges, H), jnp.float32),
    )(s, We2, be2r)

    zeros_nh = jnp.zeros((n_nodes, H), jnp.float32)
    aggp = _make_scatter_kernel(n_nodes, n_edges, eb)(ef, row3, zeros_nh)

    nb2 = n_nodes // nblk
    out = pl.pallas_call(
        _node_kernel,
        grid=(nb2,),
        in_specs=[
            pl.BlockSpec((nblk, H), lambda i: (i, 0)),
            pl.BlockSpec((nblk, H), lambda i: (i, 0)),
            pl.BlockSpec((nblk, H), lambda i, n=nb2: (i + n, 0)),
            pl.BlockSpec((H, H), lambda i: (0, 0)),
            pl.BlockSpec((H, H), lambda i: (0, 0)),
            pl.BlockSpec((1, H), lambda i: (0, 0)),
            pl.BlockSpec((H, H), lambda i: (0, 0)),
            pl.BlockSpec((1, H), lambda i: (0, 0)),
            pl.BlockSpec((1, H), lambda i: (0, 0)),
            pl.BlockSpec((1, H), lambda i: (0, 0)),
        ],
        out_specs=pl.BlockSpec((nblk, H), lambda i: (i, 0)),
        out_shape=jax.ShapeDtypeStruct((n_nodes, H), jnp.float32),
    )(x, aggp, aggp, wn1a, wn1b, bn1r, Wn2, bn2r, gr, br)

    return out
